# Initial kernel scaffold; baseline (speedup 1.0000x reference)
import jax
import jax.numpy as jnp
from jax.experimental import pallas as pl


def kernel(h, edge_index, weights, bias):
    raise NotImplementedError("write your pallas kernel here")


# speedup vs baseline: 4.9972x; 4.9972x over previous
import functools

import jax
import jax.numpy as jnp
from jax import lax
from jax.experimental import pallas as pl
from jax.experimental.pallas import tpu as pltpu
from jax.experimental.pallas import tpu_sc as plsc

N_NODES = 10000
N_EDGES = 320000
D = 128

NUM_SC = 2
NUM_TILES = 16
NUM_W = NUM_SC * NUM_TILES
EDGES_PER_W = N_EDGES // NUM_W
CHUNK = 80
NCHUNK = EDGES_PER_W // CHUNK
N_PAD = 10240
ROWS_PER_TILE = N_PAD // NUM_TILES


def _sc_aggregate(h, src, dst, zeros_tile):
    mesh = plsc.VectorSubcoreMesh(core_axis_name="c", subcore_axis_name="s")

    @functools.partial(
        pl.kernel,
        out_type=jax.ShapeDtypeStruct((NUM_SC, N_PAD, D), jnp.float32),
        mesh=mesh,
        scratch_types=[
            pltpu.VMEM((CHUNK,), jnp.int32),
            pltpu.VMEM((CHUNK,), jnp.int32),
            pltpu.VMEM((CHUNK, D), jnp.float32),
            pltpu.VMEM_SHARED((N_PAD, D), jnp.float32),
            pltpu.SemaphoreType.DMA,
        ],
    )
    def agg_kernel(h_hbm, src_hbm, dst_hbm, z_hbm, part_hbm,
                   src_v, dst_v, rows_v, acc_sh, sem):
        cid = lax.axis_index("c")
        sid = lax.axis_index("s")
        gwid = cid * NUM_TILES + sid

        pltpu.sync_copy(z_hbm, acc_sh.at[pl.ds(sid * ROWS_PER_TILE, ROWS_PER_TILE)])
        plsc.subcore_barrier()

        base = gwid * EDGES_PER_W

        def body(i, _):
            off = base + i * CHUNK
            pltpu.sync_copy(src_hbm.at[pl.ds(off, CHUNK)], src_v)
            pltpu.async_copy(h_hbm.at[src_v], rows_v, sem).wait()
            pltpu.sync_copy(dst_hbm.at[pl.ds(off, CHUNK)], dst_v)
            pltpu.sync_copy(rows_v, acc_sh.at[dst_v], add=True)
            return ()

        lax.fori_loop(0, NCHUNK, body, (), unroll=False)
        plsc.subcore_barrier()

        sl = pl.ds(sid * ROWS_PER_TILE, ROWS_PER_TILE)
        pltpu.sync_copy(acc_sh.at[sl], part_hbm.at[cid].at[sl])

    return agg_kernel(h, src, dst, zeros_tile)


def _tc_finish(partials, weights, bias2d):
    rows = 1000
    grid = N_NODES // rows
    k = weights.shape[0]

    def tc_kernel(p_ref, w_ref, b_ref, o_ref):
        p = p_ref[0] + p_ref[1]
        w = jnp.mean(w_ref[...], axis=0)
        acc = jnp.dot(p, w, preferred_element_type=jnp.float32)
        o_ref[...] = jnp.maximum(acc + b_ref[...], 0.0)

    return pl.pallas_call(
        tc_kernel,
        grid=(grid,),
        in_specs=[
            pl.BlockSpec((NUM_SC, rows, D), lambda i: (0, i, 0)),
            pl.BlockSpec((k, D, D), lambda i: (0, 0, 0)),
            pl.BlockSpec((1, D), lambda i: (0, 0)),
        ],
        out_specs=pl.BlockSpec((rows, D), lambda i: (i, 0)),
        out_shape=jax.ShapeDtypeStruct((N_NODES, D), jnp.float32),
    )(partials, weights, bias2d)


def kernel(h, edge_index, weights, bias):
    src = edge_index[0].astype(jnp.int32)
    dst = edge_index[1].astype(jnp.int32)
    h = h.astype(jnp.float32)
    zeros_tile = jnp.zeros((ROWS_PER_TILE, D), jnp.float32)
    partials = _sc_aggregate(h, src, dst, zeros_tile)
    return _tc_finish(partials, weights.astype(jnp.float32),
                      bias.astype(jnp.float32).reshape(1, D))
